# baseline (speedup 1.0000x reference)
import functools

import jax
import jax.numpy as jnp
from jax import lax
from jax.experimental import pallas as pl
from jax.experimental.pallas import tpu as pltpu
from jax.experimental.pallas import tpu_sc as plsc

N = 10000
E = 320000
D = 128

NC = 2
NS = 16
C = 128
NCK = E // C // NS
XTRA = E // C - NCK * NS
EMAX = (NCK + 1) * C
RPT = 624
REM = N - NS * RPT
RC = 104


H = D // 2


NB = 4
NRING = NCK // NB
LOOKAHEAD = 2


@functools.partial(
    pl.kernel,
    out_type=[jax.ShapeDtypeStruct((2 * N, H), jnp.float32),
              jax.ShapeDtypeStruct((2 * N, H), jnp.float32)],
    mesh=plsc.VectorSubcoreMesh(core_axis_name="c", subcore_axis_name="s"),
    compiler_params=pltpu.CompilerParams(use_tc_tiling_on_sc=False),
    scratch_types=[
        pltpu.VMEM((EMAX,), jnp.int32),
        [pltpu.VMEM((C,), jnp.int32)] * NB,
        pltpu.VMEM((EMAX,), jnp.float32),
        [pltpu.VMEM((C, H), jnp.float32)] * NB,
        pltpu.VMEM((RC, H), jnp.float32),
        pltpu.VMEM_SHARED((N, H), jnp.float32),
        [pltpu.SemaphoreType.DMA] * NB,
        [pltpu.SemaphoreType.DMA] * NB,
        [pltpu.SemaphoreType.DMA] * NB,
        pltpu.SemaphoreType.DMA,
    ],
)
def _spmm_pair(xw_lo, xw_hi, src_hbm, dst_hbm, ew_hbm, out_lo, out_hi,
               sidx, didx, wv, rows, zbuf, acc, gsem, ssem, isem, osem):
    c = lax.axis_index("c")
    s = lax.axis_index("s")
    zero16 = jnp.zeros((16,), jnp.float32)
    ebase = c * E + (NCK * s + jnp.minimum(s, XTRA)) * C

    pltpu.sync_copy(src_hbm.at[pl.ds(ebase, EMAX)], sidx)
    pltpu.sync_copy(ew_hbm.at[pl.ds(ebase, EMAX)], wv)

    def _zrow(i, carry):
        for k in range(H // 16):
            zbuf[i, pl.ds(k * 16, 16)] = zero16
        return carry
    lax.fori_loop(0, RC, _zrow, 0)

    def _scale(rows, qC):
        for g in range(C // 16):
            w16 = wv[pl.ds(qC + g * 16, 16)]
            for e in range(16):
                wspl = jnp.full((16,), w16[e], jnp.float32)
                i = g * 16 + e
                for k in range(H // 16):
                    sl = (i, pl.ds(k * 16, 16))
                    rows[sl] = rows[sl] * wspl

    for xw_h, out_h in ((xw_lo, out_lo), (xw_hi, out_hi)):
        for k in range(RPT // RC):
            pltpu.async_copy(zbuf, acc.at[pl.ds(s * RPT + k * RC, RC)], osem)
        for k in range(RPT // RC):
            pltpu.make_async_copy(zbuf, acc.at[pl.ds(0, RC)], osem).wait()

        @pl.when(s == NS - 1)
        def _zero_rem():
            pltpu.sync_copy(zbuf.at[pl.ds(0, REM)], acc.at[pl.ds(NS * RPT, REM)])
        plsc.subcore_barrier()

        for q in range(LOOKAHEAD):
            pltpu.async_copy(xw_h.at[sidx.at[pl.ds(q * C, C)]], rows[q], gsem[q])
            pltpu.async_copy(dst_hbm.at[pl.ds(ebase + q * C, C)], didx[q], isem[q])

        def _ring(i, carry):
            for u in range(NB):
                q = i * NB + u
                pltpu.make_async_copy(
                    xw_h.at[sidx.at[pl.ds(0, C)]], rows[u], gsem[u]).wait()
                _scale(rows[u], q * C)
                pltpu.make_async_copy(
                    dst_hbm.at[pl.ds(0, C)], didx[u], isem[u]).wait()
                pltpu.async_copy(rows[u], acc.at[didx[u]], ssem[u], add=True)

                bn = (u + LOOKAHEAD) % NB
                wait_ok = (u >= NB - LOOKAHEAD)
                issue_ok_last = (u < NB - LOOKAHEAD)

                def _drain():
                    pltpu.make_async_copy(
                        rows[bn], acc.at[didx[bn]], ssem[bn]).wait()

                def _issue():
                    qn = q + LOOKAHEAD
                    pltpu.async_copy(
                        xw_h.at[sidx.at[pl.ds(qn * C, C)]], rows[bn], gsem[bn])
                    pltpu.async_copy(
                        dst_hbm.at[pl.ds(ebase + qn * C, C)], didx[bn], isem[bn])

                if wait_ok:
                    _drain()
                else:
                    pl.when(i > 0)(_drain)
                if issue_ok_last:
                    _issue()
                else:
                    pl.when(i < NRING - 1)(_issue)
            return carry
        lax.fori_loop(0, NRING, _ring, 0)
        for u in range(LOOKAHEAD, NB):
            pltpu.make_async_copy(rows[u], acc.at[didx[u]], ssem[u]).wait()

        @pl.when(s < XTRA)
        def _extra():
            q = NCK
            pltpu.async_copy(
                xw_h.at[sidx.at[pl.ds(q * C, C)]], rows[0], gsem[0])
            pltpu.sync_copy(dst_hbm.at[pl.ds(ebase + q * C, C)], didx[0])
            pltpu.make_async_copy(
                xw_h.at[sidx.at[pl.ds(0, C)]], rows[0], gsem[0]).wait()
            _scale(rows[0], q * C)
            pltpu.sync_copy(rows[0], acc.at[didx[0]], add=True)
        plsc.subcore_barrier()

        pltpu.async_copy(acc.at[pl.ds(s * RPT, RPT)],
                         out_h.at[pl.ds(c * N + s * RPT, RPT)], osem)

        @pl.when(s == NS - 1)
        def _copy_rem():
            pltpu.sync_copy(acc.at[pl.ds(NS * RPT, REM)],
                            out_h.at[pl.ds(c * N + NS * RPT, REM)])
        pltpu.make_async_copy(acc.at[pl.ds(0, RPT)],
                              out_h.at[pl.ds(0, RPT)], osem).wait()
        plsc.subcore_barrier()


_BM = 1000


def _mm_body(x_ref, w_ref, o_ref):
    o_ref[...] = jnp.dot(x_ref[...], w_ref[0], preferred_element_type=jnp.float32)


def _encode_mm(x_all, w_stacked):
    nblk = (2 * N) // _BM
    return pl.pallas_call(
        _mm_body,
        grid=(nblk,),
        in_specs=[
            pl.BlockSpec((_BM, D), lambda g: (g, 0)),
            pl.BlockSpec((1, D, D), lambda g: (g // (nblk // 2), 0, 0)),
        ],
        out_specs=pl.BlockSpec((_BM, D), lambda g: (g, 0)),
        out_shape=jax.ShapeDtypeStruct((2 * N, D), jnp.float32),
    )(x_all, w_stacked)


def _fuse_body(h1_ref, h2_ref, wom_ref, u_ref, wd1_ref, wd2_ref,
               emb_ref, al_ref, z1_ref, z2_ref):
    h1 = h1_ref[...]
    h2 = h2_ref[...]
    wom = wom_ref[...]
    u = u_ref[...]
    v1 = jnp.tanh(jnp.dot(h1, wom, preferred_element_type=jnp.float32))
    v2 = jnp.tanh(jnp.dot(h2, wom, preferred_element_type=jnp.float32))
    s1 = jnp.sum(v1 * u, axis=1, keepdims=True)
    s2 = jnp.sum(v2 * u, axis=1, keepdims=True)
    m = jnp.maximum(s1, s2)
    e1 = jnp.exp(s1 - m)
    e2 = jnp.exp(s2 - m)
    inv = 1.0 / (e1 + e2)
    a1 = e1 * inv
    a2 = e2 * inv
    emb = a1 * h1 + a2 * h2
    emb_ref[...] = emb
    al_ref[...] = jnp.concatenate([a1, a2], axis=1)
    z1_ref[...] = jnp.dot(emb, wd1_ref[...], preferred_element_type=jnp.float32)
    z2_ref[...] = jnp.dot(emb, wd2_ref[...], preferred_element_type=jnp.float32)


def _fuse(h1, h2, w_omega, u_row, W_dec1, W_dec2):
    nblk = N // _BM
    full = lambda g: (0, 0)
    return pl.pallas_call(
        _fuse_body,
        grid=(nblk,),
        in_specs=[
            pl.BlockSpec((_BM, D), lambda g: (g, 0)),
            pl.BlockSpec((_BM, D), lambda g: (g, 0)),
            pl.BlockSpec((D, D), full),
            pl.BlockSpec((1, D), full),
            pl.BlockSpec((D, D), full),
            pl.BlockSpec((D, D), full),
        ],
        out_specs=[
            pl.BlockSpec((_BM, D), lambda g: (g, 0)),
            pl.BlockSpec((_BM, 2), lambda g: (g, 0)),
            pl.BlockSpec((_BM, D), lambda g: (g, 0)),
            pl.BlockSpec((_BM, D), lambda g: (g, 0)),
        ],
        out_shape=[
            jax.ShapeDtypeStruct((N, D), jnp.float32),
            jax.ShapeDtypeStruct((N, 2), jnp.float32),
            jax.ShapeDtypeStruct((N, D), jnp.float32),
            jax.ShapeDtypeStruct((N, D), jnp.float32),
        ],
    )(h1, h2, w_omega, u_row, W_dec1, W_dec2)


def kernel(x1, x2, edge_index1, edge_weight1, edge_index2, edge_weight2,
           W_enc1, W_enc2, w_omega, u_omega, W_dec1, W_dec2):
    pad_i = jnp.zeros((C,), jnp.int32)
    src_all = jnp.concatenate([edge_index1[0].astype(jnp.int32),
                               edge_index2[0].astype(jnp.int32) + N, pad_i])
    dst_all = jnp.concatenate([edge_index1[1].astype(jnp.int32),
                               edge_index2[1].astype(jnp.int32)])
    ew_all = jnp.concatenate([edge_weight1, edge_weight2,
                              jnp.zeros((C,), jnp.float32)])

    x_all = jnp.concatenate([x1, x2], axis=0)
    w_enc = jnp.stack([W_enc1, W_enc2])
    xw_all = _encode_mm(x_all, w_enc)

    h_lo, h_hi = _spmm_pair(xw_all[:, :H], xw_all[:, H:], src_all, dst_all, ew_all)
    h_all = jnp.concatenate([h_lo, h_hi], axis=1)
    h1 = h_all[:N]
    h2 = h_all[N:]

    emb, alpha, z1, z2 = _fuse(h1, h2, w_omega, u_omega.reshape(1, D),
                               W_dec1, W_dec2)

    z_all = jnp.concatenate([z1, z2], axis=0)
    d_lo, d_hi = _spmm_pair(z_all[:, :H], z_all[:, H:], src_all, dst_all, ew_all)
    d_all = jnp.concatenate([d_lo, d_hi], axis=1)
    d1 = d_all[:N]
    d2 = d_all[N:]

    return (h1, h2, emb, alpha, d1, d2)


# speedup vs baseline: 1.0321x; 1.0321x over previous
import functools

import jax
import jax.numpy as jnp
from jax import lax
from jax.experimental import pallas as pl
from jax.experimental.pallas import tpu as pltpu
from jax.experimental.pallas import tpu_sc as plsc

N = 10000
E = 320000
D = 128

NC = 2
NS = 16
C = 128
NCK = E // C // NS
XTRA = E // C - NCK * NS
EMAX = (NCK + 1) * C
RPT = 624
REM = N - NS * RPT
RC = 104


H = D // 2


NB = 3
NRING = NCK // NB
LOOKAHEAD = 2


@functools.partial(
    pl.kernel,
    out_type=[jax.ShapeDtypeStruct((2 * N, H), jnp.float32),
              jax.ShapeDtypeStruct((2 * N, H), jnp.float32)],
    mesh=plsc.VectorSubcoreMesh(core_axis_name="c", subcore_axis_name="s"),
    compiler_params=pltpu.CompilerParams(use_tc_tiling_on_sc=False),
    scratch_types=[
        pltpu.VMEM((EMAX,), jnp.int32),
        pltpu.VMEM((NCK + 1, C), jnp.int32),
        pltpu.VMEM((EMAX,), jnp.float32),
        [pltpu.VMEM((C, H), jnp.float32)] * NB,
        pltpu.VMEM_SHARED((N, H), jnp.float32),
        [pltpu.SemaphoreType.DMA] * NB,
        [pltpu.SemaphoreType.DMA] * NB,
        pltpu.SemaphoreType.DMA,
    ],
)
def _spmm_pair(xw_lo, xw_hi, src_hbm, dst2d_hbm, ew_hbm, zeros_hbm, out_lo, out_hi,
               sidx, didx, wv, rows, acc, gsem, ssem, osem):
    c = lax.axis_index("c")
    s = lax.axis_index("s")
    ebase = c * E + (NCK * s + jnp.minimum(s, XTRA)) * C

    pltpu.sync_copy(src_hbm.at[pl.ds(ebase, EMAX)], sidx)
    pltpu.sync_copy(ew_hbm.at[pl.ds(ebase, EMAX)], wv)
    cbase = c * (E // C) + NCK * s + jnp.minimum(s, XTRA)
    pltpu.sync_copy(dst2d_hbm.at[pl.ds(cbase, NCK + 1)], didx)

    def _scale(rows, qC):
        for g in range(C // 16):
            w16 = wv[pl.ds(qC + g * 16, 16)]
            for e in range(16):
                wspl = jnp.full((16,), w16[e], jnp.float32)
                i = g * 16 + e
                for k in range(H // 16):
                    sl = (i, pl.ds(k * 16, 16))
                    rows[sl] = rows[sl] * wspl

    for xw_h, out_h in ((xw_lo, out_lo), (xw_hi, out_hi)):
        pltpu.async_copy(zeros_hbm, acc.at[pl.ds(s * RPT, RPT)], osem)

        @pl.when(s == NS - 1)
        def _zero_rem():
            pltpu.sync_copy(zeros_hbm.at[pl.ds(0, REM)], acc.at[pl.ds(NS * RPT, REM)])
        pltpu.make_async_copy(zeros_hbm, acc.at[pl.ds(0, RPT)], osem).wait()
        plsc.subcore_barrier()

        for q in range(LOOKAHEAD):
            pltpu.async_copy(xw_h.at[sidx.at[pl.ds(q * C, C)]], rows[q], gsem[q])

        def _ring(i, carry):
            for u in range(NB):
                q = i * NB + u
                pltpu.make_async_copy(
                    xw_h.at[sidx.at[pl.ds(0, C)]], rows[u], gsem[u]).wait()
                _scale(rows[u], q * C)
                pltpu.async_copy(rows[u], acc.at[didx.at[q]], ssem[u], add=True)

                bn = (u + LOOKAHEAD) % NB
                wait_ok = (u >= NB - LOOKAHEAD)
                issue_ok_last = (u < NB - LOOKAHEAD)

                def _drain():
                    pltpu.make_async_copy(
                        rows[bn], acc.at[didx.at[0]], ssem[bn]).wait()

                def _issue():
                    qn = q + LOOKAHEAD
                    pltpu.async_copy(
                        xw_h.at[sidx.at[pl.ds(qn * C, C)]], rows[bn], gsem[bn])

                if wait_ok:
                    _drain()
                else:
                    pl.when(i > 0)(_drain)
                if issue_ok_last:
                    _issue()
                else:
                    pl.when(i < NRING - 1)(_issue)
            return carry
        lax.fori_loop(0, NRING, _ring, 0)
        for u in range(LOOKAHEAD, NB):
            pltpu.make_async_copy(rows[u], acc.at[didx.at[0]], ssem[u]).wait()

        @pl.when(s < XTRA)
        def _extra():
            q = NCK
            pltpu.async_copy(
                xw_h.at[sidx.at[pl.ds(q * C, C)]], rows[0], gsem[0])
            pltpu.make_async_copy(
                xw_h.at[sidx.at[pl.ds(0, C)]], rows[0], gsem[0]).wait()
            _scale(rows[0], q * C)
            pltpu.sync_copy(rows[0], acc.at[didx.at[q]], add=True)
        plsc.subcore_barrier()

        pltpu.async_copy(acc.at[pl.ds(s * RPT, RPT)],
                         out_h.at[pl.ds(c * N + s * RPT, RPT)], osem)

        @pl.when(s == NS - 1)
        def _copy_rem():
            pltpu.sync_copy(acc.at[pl.ds(NS * RPT, REM)],
                            out_h.at[pl.ds(c * N + NS * RPT, REM)])
        pltpu.make_async_copy(acc.at[pl.ds(0, RPT)],
                              out_h.at[pl.ds(0, RPT)], osem).wait()
        plsc.subcore_barrier()


_BM = 1000


def _mm_body(x_ref, w_ref, o_ref):
    o_ref[...] = jnp.dot(x_ref[...], w_ref[0], preferred_element_type=jnp.float32)


def _encode_mm(x_all, w_stacked):
    nblk = (2 * N) // _BM
    return pl.pallas_call(
        _mm_body,
        grid=(nblk,),
        in_specs=[
            pl.BlockSpec((_BM, D), lambda g: (g, 0)),
            pl.BlockSpec((1, D, D), lambda g: (g // (nblk // 2), 0, 0)),
        ],
        out_specs=pl.BlockSpec((_BM, D), lambda g: (g, 0)),
        out_shape=jax.ShapeDtypeStruct((2 * N, D), jnp.float32),
    )(x_all, w_stacked)


def _fuse_body(h1_ref, h2_ref, wom_ref, u_ref, wd1_ref, wd2_ref,
               emb_ref, al_ref, z1_ref, z2_ref):
    h1 = h1_ref[...]
    h2 = h2_ref[...]
    wom = wom_ref[...]
    u = u_ref[...]
    v1 = jnp.tanh(jnp.dot(h1, wom, preferred_element_type=jnp.float32))
    v2 = jnp.tanh(jnp.dot(h2, wom, preferred_element_type=jnp.float32))
    s1 = jnp.sum(v1 * u, axis=1, keepdims=True)
    s2 = jnp.sum(v2 * u, axis=1, keepdims=True)
    m = jnp.maximum(s1, s2)
    e1 = jnp.exp(s1 - m)
    e2 = jnp.exp(s2 - m)
    inv = 1.0 / (e1 + e2)
    a1 = e1 * inv
    a2 = e2 * inv
    emb = a1 * h1 + a2 * h2
    emb_ref[...] = emb
    al_ref[...] = jnp.concatenate([a1, a2], axis=1)
    z1_ref[...] = jnp.dot(emb, wd1_ref[...], preferred_element_type=jnp.float32)
    z2_ref[...] = jnp.dot(emb, wd2_ref[...], preferred_element_type=jnp.float32)


def _fuse(h1, h2, w_omega, u_row, W_dec1, W_dec2):
    nblk = N // _BM
    full = lambda g: (0, 0)
    return pl.pallas_call(
        _fuse_body,
        grid=(nblk,),
        in_specs=[
            pl.BlockSpec((_BM, D), lambda g: (g, 0)),
            pl.BlockSpec((_BM, D), lambda g: (g, 0)),
            pl.BlockSpec((D, D), full),
            pl.BlockSpec((1, D), full),
            pl.BlockSpec((D, D), full),
            pl.BlockSpec((D, D), full),
        ],
        out_specs=[
            pl.BlockSpec((_BM, D), lambda g: (g, 0)),
            pl.BlockSpec((_BM, 2), lambda g: (g, 0)),
            pl.BlockSpec((_BM, D), lambda g: (g, 0)),
            pl.BlockSpec((_BM, D), lambda g: (g, 0)),
        ],
        out_shape=[
            jax.ShapeDtypeStruct((N, D), jnp.float32),
            jax.ShapeDtypeStruct((N, 2), jnp.float32),
            jax.ShapeDtypeStruct((N, D), jnp.float32),
            jax.ShapeDtypeStruct((N, D), jnp.float32),
        ],
    )(h1, h2, w_omega, u_row, W_dec1, W_dec2)


def kernel(x1, x2, edge_index1, edge_weight1, edge_index2, edge_weight2,
           W_enc1, W_enc2, w_omega, u_omega, W_dec1, W_dec2):
    pad_i = jnp.zeros((C,), jnp.int32)
    src_all = jnp.concatenate([edge_index1[0].astype(jnp.int32),
                               edge_index2[0].astype(jnp.int32) + N, pad_i])
    dst_all = jnp.concatenate([edge_index1[1].astype(jnp.int32),
                               edge_index2[1].astype(jnp.int32), pad_i])
    ew_all = jnp.concatenate([edge_weight1, edge_weight2,
                              jnp.zeros((C,), jnp.float32)])

    x_all = jnp.concatenate([x1, x2], axis=0)
    w_enc = jnp.stack([W_enc1, W_enc2])
    xw_all = _encode_mm(x_all, w_enc)

    dst2d = dst_all.reshape(-1, C)
    zrows = jnp.zeros((RPT, H), jnp.float32)

    h_lo, h_hi = _spmm_pair(xw_all[:, :H], xw_all[:, H:], src_all, dst2d, ew_all, zrows)
    h_all = jnp.concatenate([h_lo, h_hi], axis=1)
    h1 = h_all[:N]
    h2 = h_all[N:]

    emb, alpha, z1, z2 = _fuse(h1, h2, w_omega, u_omega.reshape(1, D),
                               W_dec1, W_dec2)

    z_all = jnp.concatenate([z1, z2], axis=0)
    d_lo, d_hi = _spmm_pair(z_all[:, :H], z_all[:, H:], src_all, dst2d, ew_all, zrows)
    d_all = jnp.concatenate([d_lo, d_hi], axis=1)
    d1 = d_all[:N]
    d2 = d_all[N:]

    return (h1, h2, emb, alpha, d1, d2)
